# Initial kernel scaffold; baseline (speedup 1.0000x reference)
import jax
import jax.numpy as jnp
from jax.experimental import pallas as pl


def kernel(h, r, pos_t, neg_t, entity_embed, relation_embed_w, trans_M, modulus):
    raise NotImplementedError("write your pallas kernel here")


# speedup vs baseline: 4.5737x; 4.5737x over previous
import functools

import jax
import jax.numpy as jnp
from jax import lax
from jax.experimental import pallas as pl
from jax.experimental.pallas import tpu as pltpu
from jax.experimental.pallas import tpu_sc as plsc

N_ENTITIES = 100000
N_RELATIONS = 32
EMBED_DIM = 128
REL_DIM = 64
GAMMA = 12.0
EPSILON = 2.0
EMB_RANGE = (GAMMA + EPSILON) / EMBED_DIM
KG_LAMBDA = 1e-05
B = 16384
PI = 3.1415926235897933

NC = 2
NS = 16
NW = NC * NS
TOTAL_ROWS = 3 * B
ROWS_PER_W = TOTAL_ROWS // NW
CHUNK = 128
NCHUNK = ROWS_PER_W // CHUNK

T = 256
NGROUP = N_RELATIONS // 2


def _sc_gather(table, idx):
    mesh = plsc.VectorSubcoreMesh(
        core_axis_name="c", subcore_axis_name="s", num_cores=NC, num_subcores=NS)

    @functools.partial(
        pl.kernel,
        out_type=jax.ShapeDtypeStruct((TOTAL_ROWS, EMBED_DIM), jnp.float32),
        mesh=mesh,
        scratch_types=[
            pltpu.VMEM((ROWS_PER_W,), jnp.int32),
            pltpu.VMEM((CHUNK, EMBED_DIM), jnp.float32),
            pltpu.SemaphoreType.DMA,
        ],
    )
    def gather_kernel(table_hbm, idx_hbm, out_hbm, idx_v, rows_v, sem):
        wid = lax.axis_index("s") * NC + lax.axis_index("c")
        pltpu.sync_copy(idx_hbm.at[pl.ds(wid * ROWS_PER_W, ROWS_PER_W)], idx_v)
        for c in range(NCHUNK):
            pltpu.async_copy(
                table_hbm.at[idx_v.at[pl.ds(c * CHUNK, CHUNK)]], rows_v,
                sem).wait()
            pltpu.sync_copy(
                rows_v, out_hbm.at[pl.ds(wid * ROWS_PER_W + c * CHUNK, CHUNK)])

    return gather_kernel(table, idx)


def _score_kernel(h_ref, p_ref, n_ref, r_ref, w_ref, r128_ref, mod_ref,
                  kg_ref, l2_ref):
    i = pl.program_id(0)
    inv_scale = PI / EMB_RANGE

    x3 = jnp.concatenate([h_ref[...], p_ref[...], n_ref[...]], axis=0)
    r = r_ref[...]
    r3 = jnp.concatenate([r, r, r], axis=0)
    lane = lax.broadcasted_iota(jnp.int32, (3 * T, EMBED_DIM), 1)
    half = (lane >= REL_DIM).astype(jnp.int32)

    acc = jnp.zeros((3 * T, EMBED_DIM), jnp.float32)
    for g in range(NGROUP):
        wg = w_ref[:, g * 128:(g + 1) * 128]
        m = (r3 == (2 * g + half)).astype(jnp.float32)
        acc = acc + jnp.dot(x3, wg, preferred_element_type=jnp.float32) * m

    acc = acc * inv_scale
    a_h = acc[0:T]
    a_p = acc[T:2 * T]
    a_n = acc[2 * T:3 * T]

    rel32 = lax.broadcasted_iota(jnp.int32, (T, N_RELATIONS), 1)
    onehot = (r == rel32).astype(jnp.float32)
    remb = jnp.dot(onehot, r128_ref[...],
                   preferred_element_type=jnp.float32) * inv_scale

    mval = mod_ref[0, 0]
    pos_sum = jnp.sum(jnp.abs(jnp.sin(a_h + remb - a_p)), axis=1, keepdims=True)
    neg_sum = jnp.sum(jnp.abs(jnp.sin(a_h + remb - a_n)), axis=1, keepdims=True)
    pos_score = GAMMA - pos_sum * mval
    neg_score = GAMMA - neg_sum * mval
    x = neg_score - pos_score
    kg = jnp.maximum(-x, 0.0) + jnp.log1p(jnp.exp(-jnp.abs(x)))
    kg_part = jnp.sum(kg).reshape(1, 1)
    l2_part = (0.5 * (jnp.sum(a_h * a_h) + jnp.sum(remb * remb)
                      + jnp.sum(a_p * a_p) + jnp.sum(a_n * a_n))).reshape(1, 1)

    @pl.when(i == 0)
    def _():
        kg_ref[...] = jnp.zeros((1, 1), jnp.float32)
        l2_ref[...] = jnp.zeros((1, 1), jnp.float32)

    kg_ref[...] += kg_part
    l2_ref[...] += l2_part


def _tc_score(rows, r2d, w_t, r128, modulus):
    nblk = B // T
    grid_spec = pl.GridSpec(
        grid=(nblk,),
        in_specs=[
            pl.BlockSpec((T, EMBED_DIM), lambda i: (i, 0)),
            pl.BlockSpec((T, EMBED_DIM), lambda i: (i + nblk, 0)),
            pl.BlockSpec((T, EMBED_DIM), lambda i: (i + 2 * nblk, 0)),
            pl.BlockSpec((T, 1), lambda i: (i, 0)),
            pl.BlockSpec((EMBED_DIM, N_RELATIONS * REL_DIM), lambda i: (0, 0)),
            pl.BlockSpec((N_RELATIONS, EMBED_DIM), lambda i: (0, 0)),
            pl.BlockSpec((1, 1), lambda i: (0, 0)),
        ],
        out_specs=[
            pl.BlockSpec((1, 1), lambda i: (0, 0)),
            pl.BlockSpec((1, 1), lambda i: (0, 0)),
        ],
    )
    return pl.pallas_call(
        _score_kernel,
        grid_spec=grid_spec,
        out_shape=[
            jax.ShapeDtypeStruct((1, 1), jnp.float32),
            jax.ShapeDtypeStruct((1, 1), jnp.float32),
        ],
    )(rows, rows, rows, r2d, w_t, r128, modulus)


def kernel(h, r, pos_t, neg_t, entity_embed, relation_embed_w, trans_M, modulus):
    idx = jnp.concatenate([h, pos_t, neg_t]).astype(jnp.int32)
    rows = _sc_gather(entity_embed, idx)

    w_t = jnp.transpose(trans_M, (1, 0, 2)).reshape(EMBED_DIM,
                                                    N_RELATIONS * REL_DIM)
    parity = (jnp.arange(N_RELATIONS, dtype=jnp.int32) % 2)[:, None]
    parity = parity.astype(jnp.float32)
    r128 = jnp.concatenate([relation_embed_w * (1.0 - parity),
                            relation_embed_w * parity], axis=1)
    r2d = r.astype(jnp.int32).reshape(B, 1)

    kg_sum, l2_sum = _tc_score(rows, r2d, w_t, r128, modulus)
    return (kg_sum[0, 0] + KG_LAMBDA * l2_sum[0, 0]) / B


# speedup vs baseline: 4.7023x; 1.0281x over previous
import functools

import jax
import jax.numpy as jnp
from jax import lax
from jax.experimental import pallas as pl
from jax.experimental.pallas import tpu as pltpu
from jax.experimental.pallas import tpu_sc as plsc

N_ENTITIES = 100000
N_RELATIONS = 32
EMBED_DIM = 128
REL_DIM = 64
GAMMA = 12.0
EPSILON = 2.0
EMB_RANGE = (GAMMA + EPSILON) / EMBED_DIM
KG_LAMBDA = 1e-05
B = 16384
PI = 3.1415926235897933

NC = 2
NS = 16
NW = NC * NS
TOTAL_ROWS = 3 * B
ROWS_PER_W = TOTAL_ROWS // NW
CHUNK = 128
NCHUNK = ROWS_PER_W // CHUNK

T = 256
NGROUP = N_RELATIONS // 2


def _sc_gather(table, idx):
    mesh = plsc.VectorSubcoreMesh(
        core_axis_name="c", subcore_axis_name="s", num_cores=NC, num_subcores=NS)

    @functools.partial(
        pl.kernel,
        out_type=jax.ShapeDtypeStruct((TOTAL_ROWS, EMBED_DIM), jnp.float32),
        mesh=mesh,
        scratch_types=[
            pltpu.VMEM((ROWS_PER_W,), jnp.int32),
            pltpu.VMEM((CHUNK, EMBED_DIM), jnp.float32),
            pltpu.SemaphoreType.DMA,
        ],
    )
    def gather_kernel(table_hbm, idx_hbm, out_hbm, idx_v, rows_v, sem):
        wid = lax.axis_index("s") * NC + lax.axis_index("c")
        pltpu.sync_copy(idx_hbm.at[pl.ds(wid * ROWS_PER_W, ROWS_PER_W)], idx_v)
        for c in range(NCHUNK):
            pltpu.async_copy(
                table_hbm.at[idx_v.at[pl.ds(c * CHUNK, CHUNK)]], rows_v,
                sem).wait()
            pltpu.sync_copy(
                rows_v, out_hbm.at[pl.ds(wid * ROWS_PER_W + c * CHUNK, CHUNK)])

    return gather_kernel(table, idx)


def _score_kernel(h_ref, p_ref, n_ref, r_ref, w_ref, r128_ref, mod_ref,
                  kg_ref, l2_ref):
    i = pl.program_id(0)
    inv_scale = PI / EMB_RANGE

    x3 = jnp.concatenate([h_ref[...], p_ref[...], n_ref[...]], axis=0)
    x3b = x3.astype(jnp.bfloat16)
    r = r_ref[...]
    r3 = jnp.concatenate([r, r, r], axis=0)
    lane = lax.broadcasted_iota(jnp.int32, (3 * T, EMBED_DIM), 1)
    half = (lane >= REL_DIM).astype(jnp.int32)

    acc = jnp.zeros((3 * T, EMBED_DIM), jnp.float32)
    for g in range(NGROUP):
        wg = w_ref[:, g * 128:(g + 1) * 128]
        m = (r3 == (2 * g + half)).astype(jnp.float32)
        acc = acc + jnp.dot(x3b, wg, preferred_element_type=jnp.float32) * m

    acc = acc * inv_scale
    a_h = acc[0:T]
    a_p = acc[T:2 * T]
    a_n = acc[2 * T:3 * T]

    rel32 = lax.broadcasted_iota(jnp.int32, (T, N_RELATIONS), 1)
    onehot = (r == rel32).astype(jnp.float32)
    remb = jnp.dot(onehot, r128_ref[...],
                   preferred_element_type=jnp.float32) * inv_scale

    mval = mod_ref[0, 0]
    pos_sum = jnp.sum(jnp.abs(jnp.sin(a_h + remb - a_p)), axis=1, keepdims=True)
    neg_sum = jnp.sum(jnp.abs(jnp.sin(a_h + remb - a_n)), axis=1, keepdims=True)
    pos_score = GAMMA - pos_sum * mval
    neg_score = GAMMA - neg_sum * mval
    x = neg_score - pos_score
    kg = jnp.maximum(-x, 0.0) + jnp.log1p(jnp.exp(-jnp.abs(x)))
    kg_part = jnp.sum(kg).reshape(1, 1)
    l2_part = (0.5 * (jnp.sum(a_h * a_h) + jnp.sum(remb * remb)
                      + jnp.sum(a_p * a_p) + jnp.sum(a_n * a_n))).reshape(1, 1)

    @pl.when(i == 0)
    def _():
        kg_ref[...] = jnp.zeros((1, 1), jnp.float32)
        l2_ref[...] = jnp.zeros((1, 1), jnp.float32)

    kg_ref[...] += kg_part
    l2_ref[...] += l2_part


def _tc_score(rows, r2d, w_t, r128, modulus):
    nblk = B // T
    grid_spec = pl.GridSpec(
        grid=(nblk,),
        in_specs=[
            pl.BlockSpec((T, EMBED_DIM), lambda i: (i, 0)),
            pl.BlockSpec((T, EMBED_DIM), lambda i: (i + nblk, 0)),
            pl.BlockSpec((T, EMBED_DIM), lambda i: (i + 2 * nblk, 0)),
            pl.BlockSpec((T, 1), lambda i: (i, 0)),
            pl.BlockSpec((EMBED_DIM, N_RELATIONS * REL_DIM), lambda i: (0, 0)),
            pl.BlockSpec((N_RELATIONS, EMBED_DIM), lambda i: (0, 0)),
            pl.BlockSpec((1, 1), lambda i: (0, 0)),
        ],
        out_specs=[
            pl.BlockSpec((1, 1), lambda i: (0, 0)),
            pl.BlockSpec((1, 1), lambda i: (0, 0)),
        ],
    )
    return pl.pallas_call(
        _score_kernel,
        grid_spec=grid_spec,
        out_shape=[
            jax.ShapeDtypeStruct((1, 1), jnp.float32),
            jax.ShapeDtypeStruct((1, 1), jnp.float32),
        ],
    )(rows, rows, rows, r2d, w_t, r128, modulus)


def kernel(h, r, pos_t, neg_t, entity_embed, relation_embed_w, trans_M, modulus):
    idx = jnp.concatenate([h, pos_t, neg_t]).astype(jnp.int32)
    rows = _sc_gather(entity_embed, idx)

    w_t = jnp.transpose(trans_M, (1, 0, 2)).reshape(
        EMBED_DIM, N_RELATIONS * REL_DIM).astype(jnp.bfloat16)
    parity = (jnp.arange(N_RELATIONS, dtype=jnp.int32) % 2)[:, None]
    parity = parity.astype(jnp.float32)
    r128 = jnp.concatenate([relation_embed_w * (1.0 - parity),
                            relation_embed_w * parity], axis=1)
    r2d = r.astype(jnp.int32).reshape(B, 1)

    kg_sum, l2_sum = _tc_score(rows, r2d, w_t, r128, modulus)
    return (kg_sum[0, 0] + KG_LAMBDA * l2_sum[0, 0]) / B


# speedup vs baseline: 5.0496x; 1.0739x over previous
import functools

import jax
import jax.numpy as jnp
from jax import lax
from jax.experimental import pallas as pl
from jax.experimental.pallas import tpu as pltpu
from jax.experimental.pallas import tpu_sc as plsc

N_ENTITIES = 100000
N_RELATIONS = 32
EMBED_DIM = 128
REL_DIM = 64
GAMMA = 12.0
EPSILON = 2.0
EMB_RANGE = (GAMMA + EPSILON) / EMBED_DIM
KG_LAMBDA = 1e-05
B = 16384
PI = 3.1415926235897933

NC = 2
NS = 16
NW = NC * NS
TOTAL_ROWS = 3 * B
ROWS_PER_W = TOTAL_ROWS // NW
CHUNK = 128
NCHUNK = ROWS_PER_W // CHUNK

T = 256
NGROUP = N_RELATIONS // 2


def _sc_gather(table, idx):
    mesh = plsc.VectorSubcoreMesh(
        core_axis_name="c", subcore_axis_name="s", num_cores=NC, num_subcores=NS)

    @functools.partial(
        pl.kernel,
        out_type=jax.ShapeDtypeStruct((TOTAL_ROWS, EMBED_DIM), jnp.float32),
        mesh=mesh,
        scratch_types=[
            pltpu.VMEM((ROWS_PER_W,), jnp.int32),
            pltpu.VMEM((CHUNK, EMBED_DIM), jnp.float32),
            pltpu.SemaphoreType.DMA,
        ],
    )
    def gather_kernel(table_hbm, idx_hbm, out_hbm, idx_v, rows_v, sem):
        wid = lax.axis_index("s") * NC + lax.axis_index("c")
        pltpu.sync_copy(idx_hbm.at[pl.ds(wid * ROWS_PER_W, ROWS_PER_W)], idx_v)
        for c in range(NCHUNK):
            pltpu.async_copy(
                table_hbm.at[idx_v.at[pl.ds(c * CHUNK, CHUNK)]], rows_v,
                sem).wait()
            pltpu.sync_copy(
                rows_v, out_hbm.at[pl.ds(wid * ROWS_PER_W + c * CHUNK, CHUNK)])

    return gather_kernel(table, idx)


def _score_kernel(h_ref, p_ref, n_ref, r_ref, w_ref, r128_ref, mod_ref,
                  kg_ref, l2_ref):
    i = pl.program_id(0)
    inv_scale = PI / EMB_RANGE

    x3 = jnp.concatenate([h_ref[...], p_ref[...], n_ref[...]], axis=0)
    x3b = x3.astype(jnp.bfloat16)
    r = r_ref[...]
    r3 = jnp.concatenate([r, r, r], axis=0)
    lane = lax.broadcasted_iota(jnp.int32, (3 * T, EMBED_DIM), 1)
    half = (lane >= REL_DIM).astype(jnp.int32)

    acc = jnp.zeros((3 * T, EMBED_DIM), jnp.float32)
    for g in range(NGROUP):
        wg = w_ref[:, g * 128:(g + 1) * 128]
        m = r3 == (2 * g + half)
        acc = jnp.where(
            m, jnp.dot(x3b, wg, preferred_element_type=jnp.float32), acc)

    acc = acc * inv_scale
    a_h = acc[0:T]
    a_p = acc[T:2 * T]
    a_n = acc[2 * T:3 * T]

    rel32 = lax.broadcasted_iota(jnp.int32, (T, N_RELATIONS), 1)
    onehot = (r == rel32).astype(jnp.float32)
    remb = jnp.dot(onehot, r128_ref[...],
                   preferred_element_type=jnp.float32) * inv_scale

    mval = mod_ref[0, 0]
    base = a_h + remb
    argp = base - a_p
    argn = base - a_n
    packed = jnp.concatenate(
        [argp[:, :REL_DIM] + argp[:, REL_DIM:],
         argn[:, :REL_DIM] + argn[:, REL_DIM:]], axis=1)
    s = jnp.abs(jnp.sin(packed))
    pos_sum = jnp.sum(s[:, :REL_DIM], axis=1, keepdims=True)
    neg_sum = jnp.sum(s[:, REL_DIM:], axis=1, keepdims=True)
    pos_score = GAMMA - pos_sum * mval
    neg_score = GAMMA - neg_sum * mval
    x = neg_score - pos_score
    kg = jnp.maximum(-x, 0.0) + jnp.log1p(jnp.exp(-jnp.abs(x)))
    kg_part = jnp.sum(kg).reshape(1, 1)
    l2_part = (0.5 * (jnp.sum(a_h * a_h) + jnp.sum(remb * remb)
                      + jnp.sum(a_p * a_p) + jnp.sum(a_n * a_n))).reshape(1, 1)

    @pl.when(i == 0)
    def _():
        kg_ref[...] = jnp.zeros((1, 1), jnp.float32)
        l2_ref[...] = jnp.zeros((1, 1), jnp.float32)

    kg_ref[...] += kg_part
    l2_ref[...] += l2_part


def _tc_score(rows, r2d, w_t, r128, modulus):
    nblk = B // T
    grid_spec = pl.GridSpec(
        grid=(nblk,),
        in_specs=[
            pl.BlockSpec((T, EMBED_DIM), lambda i: (i, 0)),
            pl.BlockSpec((T, EMBED_DIM), lambda i: (i + nblk, 0)),
            pl.BlockSpec((T, EMBED_DIM), lambda i: (i + 2 * nblk, 0)),
            pl.BlockSpec((T, 1), lambda i: (i, 0)),
            pl.BlockSpec((EMBED_DIM, N_RELATIONS * REL_DIM), lambda i: (0, 0)),
            pl.BlockSpec((N_RELATIONS, EMBED_DIM), lambda i: (0, 0)),
            pl.BlockSpec((1, 1), lambda i: (0, 0)),
        ],
        out_specs=[
            pl.BlockSpec((1, 1), lambda i: (0, 0)),
            pl.BlockSpec((1, 1), lambda i: (0, 0)),
        ],
    )
    return pl.pallas_call(
        _score_kernel,
        grid_spec=grid_spec,
        out_shape=[
            jax.ShapeDtypeStruct((1, 1), jnp.float32),
            jax.ShapeDtypeStruct((1, 1), jnp.float32),
        ],
    )(rows, rows, rows, r2d, w_t, r128, modulus)


def kernel(h, r, pos_t, neg_t, entity_embed, relation_embed_w, trans_M, modulus):
    idx = jnp.concatenate([h, pos_t, neg_t]).astype(jnp.int32)
    rows = _sc_gather(entity_embed, idx)

    w_t = jnp.transpose(trans_M, (1, 0, 2)).reshape(
        EMBED_DIM, N_RELATIONS * REL_DIM).astype(jnp.bfloat16)
    parity = (jnp.arange(N_RELATIONS, dtype=jnp.int32) % 2)[:, None]
    parity = parity.astype(jnp.float32)
    r128 = jnp.concatenate([relation_embed_w * (1.0 - parity),
                            relation_embed_w * parity], axis=1)
    r2d = r.astype(jnp.int32).reshape(B, 1)

    kg_sum, l2_sum = _tc_score(rows, r2d, w_t, r128, modulus)
    return (kg_sum[0, 0] + KG_LAMBDA * l2_sum[0, 0]) / B


# speedup vs baseline: 6.2180x; 1.2314x over previous
import functools

import jax
import jax.numpy as jnp
from jax import lax
from jax.experimental import pallas as pl
from jax.experimental.pallas import tpu as pltpu
from jax.experimental.pallas import tpu_sc as plsc

N_ENTITIES = 100000
N_RELATIONS = 32
EMBED_DIM = 128
REL_DIM = 64
GAMMA = 12.0
EPSILON = 2.0
EMB_RANGE = (GAMMA + EPSILON) / EMBED_DIM
KG_LAMBDA = 1e-05
B = 16384
PI = 3.1415926235897933

NC = 2
NS = 16
NW = NC * NS
TOTAL_ROWS = 3 * B
ROWS_PER_W = TOTAL_ROWS // NW
CHUNK = 128
NCHUNK = ROWS_PER_W // CHUNK

T = 512
NGROUP = N_RELATIONS // 2


def _sc_gather(table, idx):
    mesh = plsc.VectorSubcoreMesh(
        core_axis_name="c", subcore_axis_name="s", num_cores=NC, num_subcores=NS)

    @functools.partial(
        pl.kernel,
        out_type=jax.ShapeDtypeStruct((TOTAL_ROWS, EMBED_DIM), jnp.float32),
        mesh=mesh,
        scratch_types=[
            pltpu.VMEM((ROWS_PER_W,), jnp.int32),
            pltpu.VMEM((CHUNK, EMBED_DIM), jnp.float32),
            pltpu.SemaphoreType.DMA,
        ],
    )
    def gather_kernel(table_hbm, idx_hbm, out_hbm, idx_v, rows_v, sem):
        wid = lax.axis_index("s") * NC + lax.axis_index("c")
        pltpu.sync_copy(idx_hbm.at[pl.ds(wid * ROWS_PER_W, ROWS_PER_W)], idx_v)
        for c in range(NCHUNK):
            pltpu.async_copy(
                table_hbm.at[idx_v.at[pl.ds(c * CHUNK, CHUNK)]], rows_v,
                sem).wait()
            pltpu.sync_copy(
                rows_v, out_hbm.at[pl.ds(wid * ROWS_PER_W + c * CHUNK, CHUNK)])

    return gather_kernel(table, idx)


def _score_kernel(h_ref, p_ref, n_ref, r_ref, w_ref, r128_ref, mod_ref,
                  kg_ref, l2_ref):
    i = pl.program_id(0)
    inv_scale = PI / EMB_RANGE

    x3 = jnp.concatenate([h_ref[...], p_ref[...], n_ref[...]], axis=0)
    x3b = x3.astype(jnp.bfloat16)
    r = r_ref[...]
    r3 = jnp.concatenate([r, r, r], axis=0)
    lane = lax.broadcasted_iota(jnp.int32, (3 * T, EMBED_DIM), 1)
    half = (lane >= REL_DIM).astype(jnp.int32)

    acc = jnp.zeros((3 * T, EMBED_DIM), jnp.float32)
    for g in range(NGROUP):
        wg = w_ref[:, g * 128:(g + 1) * 128]
        m = r3 == (2 * g + half)
        acc = jnp.where(
            m, jnp.dot(x3b, wg, preferred_element_type=jnp.float32), acc)

    acc = acc * inv_scale
    a_h = acc[0:T]
    a_p = acc[T:2 * T]
    a_n = acc[2 * T:3 * T]

    rel32 = lax.broadcasted_iota(jnp.int32, (T, N_RELATIONS), 1)
    onehot = (r == rel32).astype(jnp.float32)
    remb = jnp.dot(onehot, r128_ref[...],
                   preferred_element_type=jnp.float32) * inv_scale

    mval = mod_ref[0, 0]
    base = a_h + remb
    argp = base - a_p
    argn = base - a_n
    packed = jnp.concatenate(
        [argp[:, :REL_DIM] + argp[:, REL_DIM:],
         argn[:, :REL_DIM] + argn[:, REL_DIM:]], axis=1)
    u = packed * (1.0 / PI)
    f = u - jnp.round(u)
    y = jnp.abs(f)
    y2 = y * y
    s = y * (3.14158476 + y2 * (-5.16724806 + y2 * (2.54287504
                                                    + y2 * -0.55715812)))
    pos_sum = jnp.sum(s[:, :REL_DIM], axis=1, keepdims=True)
    neg_sum = jnp.sum(s[:, REL_DIM:], axis=1, keepdims=True)
    pos_score = GAMMA - pos_sum * mval
    neg_score = GAMMA - neg_sum * mval
    x = neg_score - pos_score
    kg = jnp.maximum(-x, 0.0) + jnp.log1p(jnp.exp(-jnp.abs(x)))
    kg_part = jnp.sum(kg).reshape(1, 1)
    l2_part = (0.5 * (jnp.sum(a_h * a_h) + jnp.sum(remb * remb)
                      + jnp.sum(a_p * a_p) + jnp.sum(a_n * a_n))).reshape(1, 1)

    @pl.when(i == 0)
    def _():
        kg_ref[...] = jnp.zeros((1, 1), jnp.float32)
        l2_ref[...] = jnp.zeros((1, 1), jnp.float32)

    kg_ref[...] += kg_part
    l2_ref[...] += l2_part


def _tc_score(rows, r2d, w_t, r128, modulus):
    nblk = B // T
    grid_spec = pl.GridSpec(
        grid=(nblk,),
        in_specs=[
            pl.BlockSpec((T, EMBED_DIM), lambda i: (i, 0)),
            pl.BlockSpec((T, EMBED_DIM), lambda i: (i + nblk, 0)),
            pl.BlockSpec((T, EMBED_DIM), lambda i: (i + 2 * nblk, 0)),
            pl.BlockSpec((T, 1), lambda i: (i, 0)),
            pl.BlockSpec((EMBED_DIM, N_RELATIONS * REL_DIM), lambda i: (0, 0)),
            pl.BlockSpec((N_RELATIONS, EMBED_DIM), lambda i: (0, 0)),
            pl.BlockSpec((1, 1), lambda i: (0, 0)),
        ],
        out_specs=[
            pl.BlockSpec((1, 1), lambda i: (0, 0)),
            pl.BlockSpec((1, 1), lambda i: (0, 0)),
        ],
    )
    return pl.pallas_call(
        _score_kernel,
        grid_spec=grid_spec,
        out_shape=[
            jax.ShapeDtypeStruct((1, 1), jnp.float32),
            jax.ShapeDtypeStruct((1, 1), jnp.float32),
        ],
    )(rows, rows, rows, r2d, w_t, r128, modulus)


def kernel(h, r, pos_t, neg_t, entity_embed, relation_embed_w, trans_M, modulus):
    idx = jnp.concatenate([h, pos_t, neg_t]).astype(jnp.int32)
    rows = _sc_gather(entity_embed, idx)

    w_t = jnp.transpose(trans_M, (1, 0, 2)).reshape(
        EMBED_DIM, N_RELATIONS * REL_DIM).astype(jnp.bfloat16)
    parity = (jnp.arange(N_RELATIONS, dtype=jnp.int32) % 2)[:, None]
    parity = parity.astype(jnp.float32)
    r128 = jnp.concatenate([relation_embed_w * (1.0 - parity),
                            relation_embed_w * parity], axis=1)
    r2d = r.astype(jnp.int32).reshape(B, 1)

    kg_sum, l2_sum = _tc_score(rows, r2d, w_t, r128, modulus)
    return (kg_sum[0, 0] + KG_LAMBDA * l2_sum[0, 0]) / B


# speedup vs baseline: 6.8554x; 1.1025x over previous
import functools

import jax
import jax.numpy as jnp
from jax import lax
from jax.experimental import pallas as pl
from jax.experimental.pallas import tpu as pltpu
from jax.experimental.pallas import tpu_sc as plsc

N_ENTITIES = 100000
N_RELATIONS = 32
EMBED_DIM = 128
REL_DIM = 64
GAMMA = 12.0
EPSILON = 2.0
EMB_RANGE = (GAMMA + EPSILON) / EMBED_DIM
KG_LAMBDA = 1e-05
B = 16384
PI = 3.1415926235897933

NC = 2
NS = 16
NW = NC * NS
TOTAL_ROWS = 3 * B
ROWS_PER_W = TOTAL_ROWS // NW
CHUNK = 128
NCHUNK = ROWS_PER_W // CHUNK

T = 512
NGROUP = N_RELATIONS // 2


def _sc_gather(table, idx):
    mesh = plsc.VectorSubcoreMesh(
        core_axis_name="c", subcore_axis_name="s", num_cores=NC, num_subcores=NS)

    @functools.partial(
        pl.kernel,
        out_type=jax.ShapeDtypeStruct((TOTAL_ROWS, EMBED_DIM), jnp.float32),
        mesh=mesh,
        scratch_types=[
            pltpu.VMEM((ROWS_PER_W,), jnp.int32),
            pltpu.VMEM((CHUNK, EMBED_DIM), jnp.float32),
            pltpu.SemaphoreType.DMA,
        ],
    )
    def gather_kernel(table_hbm, idx_hbm, out_hbm, idx_v, rows_v, sem):
        wid = lax.axis_index("s") * NC + lax.axis_index("c")
        pltpu.sync_copy(idx_hbm.at[pl.ds(wid * ROWS_PER_W, ROWS_PER_W)], idx_v)
        for c in range(NCHUNK):
            pltpu.async_copy(
                table_hbm.at[idx_v.at[pl.ds(c * CHUNK, CHUNK)]], rows_v,
                sem).wait()
            pltpu.sync_copy(
                rows_v, out_hbm.at[pl.ds(wid * ROWS_PER_W + c * CHUNK, CHUNK)])

    return gather_kernel(table, idx)


def _score_kernel(h_ref, p_ref, n_ref, r_ref, w_ref, r128_ref, mod_ref,
                  kg_ref, l2_ref):
    i = pl.program_id(0)
    inv_scale = PI / EMB_RANGE

    x3 = jnp.concatenate([h_ref[...], p_ref[...], n_ref[...]], axis=0)
    x3b = x3.astype(jnp.bfloat16)
    r = r_ref[...]
    r3 = jnp.concatenate([r, r, r], axis=0)
    lane = lax.broadcasted_iota(jnp.int32, (3 * T, EMBED_DIM), 1)
    half = (lane >= REL_DIM).astype(jnp.int32)

    acc = jnp.zeros((3 * T, EMBED_DIM), jnp.float32)
    for g in range(NGROUP):
        wg = w_ref[:, g * 128:(g + 1) * 128]
        m = r3 == (2 * g + half)
        acc = jnp.where(
            m, jnp.dot(x3b, wg, preferred_element_type=jnp.float32), acc)

    acc = acc * inv_scale
    a_h = acc[0:T]
    a_p = acc[T:2 * T]
    a_n = acc[2 * T:3 * T]

    rel32 = lax.broadcasted_iota(jnp.int32, (T, N_RELATIONS), 1)
    onehot = (r == rel32).astype(jnp.float32)
    remb = jnp.dot(onehot, r128_ref[...],
                   preferred_element_type=jnp.float32) * inv_scale

    mval = mod_ref[0, 0]
    base = a_h + remb
    argp = base - a_p
    argn = base - a_n
    packed = jnp.concatenate(
        [argp[:, :REL_DIM] + argp[:, REL_DIM:],
         argn[:, :REL_DIM] + argn[:, REL_DIM:]], axis=1)
    u = packed * (1.0 / PI)
    f = u - jnp.round(u)
    y = jnp.abs(f)
    y2 = y * y
    s = y * (3.14158476 + y2 * (-5.16724806 + y2 * (2.54287504
                                                    + y2 * -0.55715812)))
    lane2 = lax.broadcasted_iota(jnp.int32, (T, EMBED_DIM), 1)
    sgn = jnp.where(lane2 < REL_DIM, 1.0, -1.0)
    d = jnp.sum(s * sgn, axis=1, keepdims=True)
    d4 = d.reshape(T // 128, 128)
    x = d4 * mval
    kg = jnp.maximum(-x, 0.0) + jnp.log1p(jnp.exp(-jnp.abs(x)))
    kg_part = jnp.sum(kg).reshape(1, 1)
    l2_part = (0.5 * (jnp.sum(a_h * a_h) + jnp.sum(remb * remb)
                      + jnp.sum(a_p * a_p) + jnp.sum(a_n * a_n))).reshape(1, 1)

    @pl.when(i == 0)
    def _():
        kg_ref[...] = jnp.zeros((1, 1), jnp.float32)
        l2_ref[...] = jnp.zeros((1, 1), jnp.float32)

    kg_ref[...] += kg_part
    l2_ref[...] += l2_part


def _tc_score(rows, r2d, w_t, r128, modulus):
    nblk = B // T
    grid_spec = pl.GridSpec(
        grid=(nblk,),
        in_specs=[
            pl.BlockSpec((T, EMBED_DIM), lambda i: (i, 0)),
            pl.BlockSpec((T, EMBED_DIM), lambda i: (i + nblk, 0)),
            pl.BlockSpec((T, EMBED_DIM), lambda i: (i + 2 * nblk, 0)),
            pl.BlockSpec((T, 1), lambda i: (i, 0)),
            pl.BlockSpec((EMBED_DIM, N_RELATIONS * REL_DIM), lambda i: (0, 0)),
            pl.BlockSpec((N_RELATIONS, EMBED_DIM), lambda i: (0, 0)),
            pl.BlockSpec((1, 1), lambda i: (0, 0)),
        ],
        out_specs=[
            pl.BlockSpec((1, 1), lambda i: (0, 0)),
            pl.BlockSpec((1, 1), lambda i: (0, 0)),
        ],
    )
    return pl.pallas_call(
        _score_kernel,
        grid_spec=grid_spec,
        out_shape=[
            jax.ShapeDtypeStruct((1, 1), jnp.float32),
            jax.ShapeDtypeStruct((1, 1), jnp.float32),
        ],
    )(rows, rows, rows, r2d, w_t, r128, modulus)


def kernel(h, r, pos_t, neg_t, entity_embed, relation_embed_w, trans_M, modulus):
    idx = jnp.concatenate([h, pos_t, neg_t]).astype(jnp.int32)
    rows = _sc_gather(entity_embed, idx)

    w_t = jnp.transpose(trans_M, (1, 0, 2)).reshape(
        EMBED_DIM, N_RELATIONS * REL_DIM).astype(jnp.bfloat16)
    parity = (jnp.arange(N_RELATIONS, dtype=jnp.int32) % 2)[:, None]
    parity = parity.astype(jnp.float32)
    r128 = jnp.concatenate([relation_embed_w * (1.0 - parity),
                            relation_embed_w * parity], axis=1)
    r2d = r.astype(jnp.int32).reshape(B, 1)

    kg_sum, l2_sum = _tc_score(rows, r2d, w_t, r128, modulus)
    return (kg_sum[0, 0] + KG_LAMBDA * l2_sum[0, 0]) / B
